# baseline (speedup 1.0000x reference)
import functools

import jax
import jax.numpy as jnp
from jax import lax
from jax.experimental import pallas as pl
from jax.experimental.pallas import tpu as pltpu
from jax.experimental.pallas import tpu_sc as plsc


_K = 9
_S = 1 << _K
_V = 10
_N = 18
_NC = 2
_NS = 16
_NW = _NC * _NS
_L = 16
_AG = 8
_BG = 4
_APW = _S // _AG
_BPW = _S // _BG
_NVW = _BPW // _L
_PR = 248

_f32 = jnp.float32
_i32 = jnp.int32


def _tree(terms):
    terms = list(terms)
    while len(terms) > 1:
        nxt = [terms[i] + terms[i + 1] for i in range(0, len(terms) - 1, 2)]
        if len(terms) % 2:
            nxt.append(terms[-1])
        terms = nxt
    return terms[0]


def _gtake(x, idx):
    return x.at[idx].get(mode="promise_in_bounds")


def _bsum(x, lanes):
    for st in (1, 2, 4, 8):
        x = x + _gtake(x, lanes ^ st)
    return x


def _masked_sum(rows, bits, zero):
    terms = []
    for r, m in zip(rows, bits):
        if m is None:
            continue
        terms.append(r if isinstance(m, int) else m * r)
    return _tree(terms) if terms else zero


def _sc_body(params_hbm, out_hbm, pv, ebv, cacc, eat, gtab, obuf):
    c = lax.axis_index("c")
    s = lax.axis_index("s")
    wid = s * _NC + c
    ag = wid // _BG
    bg = wid - ag * _BG
    a_base = ag * _APW

    pltpu.sync_copy(params_hbm, pv)

    lanes = lax.iota(_i32, _L)
    zero = jnp.zeros((_L,), _f32)

    def _lmask(j):
        return jnp.where(((lanes >> j) & 1) == 1, 1.0, 0.0).astype(_f32)

    def _cmask(val, t):
        return jnp.where(((jnp.full((_L,), val, _i32) >> t) & 1) == 1,
                         1.0, 0.0).astype(_f32)

    lmf = [_lmask(j) for j in range(4)]
    wmf = [_cmask(bg, t) for t in range(2)]
    amf = [_cmask(ag, t) for t in range(3)]

    def _bitmask(mid_n, idx, top):
        out = list(lmf)
        for t in range(mid_n):
            out.append(1 if (idx >> t) & 1 else None)
        out.extend(top)
        return out

    for u in range(_NVW):
        bf = _bitmask(3, u, wmf)
        s_rows = [_masked_sum([pv[162 + i * _K + j] for j in range(_K)],
                              bf, zero) for i in range(_K)]
        ebv[pl.ds(u * _L, _L)] = _masked_sum(s_rows, bf, zero)
        cacc[pl.ds(u * _L, _L)] = zero

    for g in range(_APW // _L):
        af = _bitmask(2, g, amf)
        s_rows = [_masked_sum([pv[81 + i * _K + j] for j in range(_K)],
                              af, zero) for i in range(_K)]
        eat[pl.ds(g * _L, _L)] = _masked_sum(s_rows, af, zero)
        for j in range(_K):
            gtab[j, pl.ds(g * _L, _L)] = _masked_sum(
                [pv[i * _K + j] for i in range(_K)], af, zero)

    def arow(al, carry):
        plo, zvec = carry
        grp = (al // _L) * _L
        off = jnp.full((_L,), al - grp, _i32)
        ea_b = _gtake(eat[pl.ds(grp, _L)], off)
        gb = [_gtake(gtab[j, pl.ds(grp, _L)], off) for j in range(_K)]
        base = _tree([ea_b, wmf[0] * gb[7], wmf[1] * gb[8]])
        eac = [_tree([base]
                     + [gb[4 + t] for t in range(3) if (u >> t) & 1])
               for u in range(_NVW)]
        rs0 = zero
        rs1 = zero
        for u in range(_NVW):
            t = _tree([ebv[pl.ds(u * _L, _L)] + eac[u]]
                      + [lmf[j] * gb[j] for j in range(4)])
            w = jnp.exp(t)
            cacc[pl.ds(u * _L, _L)] = cacc[pl.ds(u * _L, _L)] + w
            if u % 2 == 0:
                rs0 = rs0 + w
            else:
                rs1 = rs1 + w
        rsb = _bsum(rs0 + rs1, lanes)
        zvec = zvec + rsb
        a_full = jnp.full((_L,), a_base + al, _i32)
        a_bits = jnp.where(((a_full >> lanes) & 1) == 1, 1.0, 0.0).astype(_f32)
        return plo + rsb * a_bits, zvec

    plo, zvec = lax.fori_loop(0, _APW, arow, (zero, zero))

    cs = [cacc[pl.ds(u * _L, _L)] for u in range(_NVW)]
    call = _tree(cs)
    phi = zero
    for j in range(4):
        pj = _bsum(lmf[j] * call, lanes)
        phi = phi + pj * jnp.where(lanes == j, 1.0, 0.0).astype(_f32)
    for t in range(3):
        sub = [cs[u] for u in range(_NVW) if (u >> t) & 1]
        pj = _bsum(_tree(sub), lanes)
        phi = phi + pj * jnp.where(lanes == 4 + t, 1.0, 0.0).astype(_f32)
    for t in range(2):
        pj = _bsum(wmf[t] * call, lanes)
        phi = phi + pj * jnp.where(lanes == 7 + t, 1.0, 0.0).astype(_f32)

    obuf[pl.ds(0, _L)] = plo
    obuf[pl.ds(_L, _L)] = phi
    obuf[pl.ds(2 * _L, _L)] = zvec
    pltpu.sync_copy(obuf, out_hbm.at[wid])


_mesh = plsc.VectorSubcoreMesh(core_axis_name="c", subcore_axis_name="s",
                               num_cores=_NC, num_subcores=_NS)

_sc_call = functools.partial(
    pl.kernel,
    out_type=jax.ShapeDtypeStruct((_NW, 3 * _L), _f32),
    mesh=_mesh,
    scratch_types=[
        pltpu.VMEM((_PR, _L), _f32),
        pltpu.VMEM((_BPW,), _f32),
        pltpu.VMEM((_BPW,), _f32),
        pltpu.VMEM((_APW,), _f32),
        pltpu.VMEM((_K, _APW), _f32),
        pltpu.VMEM((3 * _L,), _f32),
    ],
)(_sc_body)


def kernel(matrix, beta):
    m = beta * matrix.astype(_f32)
    flat = jnp.concatenate([
        (2.0 * m[:_K, _K:]).reshape(_K * _K),
        m[:_K, :_K].reshape(_K * _K),
        m[_K:, _K:].reshape(_K * _K),
        jnp.zeros((_PR - 3 * _K * _K,), _f32),
    ])
    pb = jnp.broadcast_to(flat[:, None], (_PR, _L))
    out = _sc_call(pb)
    plo = jnp.sum(out[:, :_L], axis=0)
    phi = jnp.sum(out[:, _L:2 * _L], axis=0)
    z = jnp.sum(out[:, 2 * _L])
    prob = jnp.concatenate([plo[:_K], phi[:_K]]) / z
    return prob[None, :_V], prob[None, _V:_N]


# speedup vs baseline: 1.1212x; 1.1212x over previous
import functools

import jax
import jax.numpy as jnp
from jax import lax
from jax.experimental import pallas as pl
from jax.experimental.pallas import tpu as pltpu
from jax.experimental.pallas import tpu_sc as plsc


_K = 9
_S = 1 << _K
_V = 10
_N = 18
_NC = 2
_NS = 16
_NW = _NC * _NS
_L = 16
_AG = 8
_BG = 4
_APW = _S // _AG
_BPW = _S // _BG
_NVW = _BPW // _L
_PR = 248

_f32 = jnp.float32
_i32 = jnp.int32


def _tree(terms):
    terms = list(terms)
    while len(terms) > 1:
        nxt = [terms[i] + terms[i + 1] for i in range(0, len(terms) - 1, 2)]
        if len(terms) % 2:
            nxt.append(terms[-1])
        terms = nxt
    return terms[0]


def _gtake(x, idx):
    return x.at[idx].get(mode="promise_in_bounds")


def _bsum(x, lanes):
    for st in (1, 2, 4, 8):
        x = x + _gtake(x, lanes ^ st)
    return x


def _bits(vec):
    return [jnp.where(((vec >> j) & 1) == 1, 1.0, 0.0).astype(_f32)
            for j in range(_K)]


def _quadform(pv, base, bf):
    return _tree([bf[i] * _tree([pv[base + i * _K + j] * bf[j]
                                 for j in range(_K)])
                  for i in range(_K)])


def _sc_body(params_hbm, out_hbm, pv, ebv, cacc, eat, gtab, obuf):
    c = lax.axis_index("c")
    s = lax.axis_index("s")
    wid = s * _NC + c
    ag = wid // _BG
    bg = wid - ag * _BG
    a_base = ag * _APW
    b_base = bg * _BPW

    pltpu.sync_copy(params_hbm, pv)

    lanes = lax.iota(_i32, _L)
    zero = jnp.zeros((_L,), _f32)

    def _lmask(j):
        return jnp.where(((lanes >> j) & 1) == 1, 1.0, 0.0).astype(_f32)

    def _wmask(t):
        return jnp.where(((jnp.full((_L,), bg, _i32) >> t) & 1) == 1,
                         1.0, 0.0).astype(_f32)

    def build_v(v, carry):
        bvec = b_base + v * _L + lanes
        bf = _bits(bvec)
        ebv[pl.ds(v * _L, _L)] = _quadform(pv, 162, bf)
        cacc[pl.ds(v * _L, _L)] = zero
        return carry

    lax.fori_loop(0, _NVW, build_v, 0)

    def build_a(g, carry):
        avec = a_base + g * _L + lanes
        af = _bits(avec)
        eat[pl.ds(g * _L, _L)] = _quadform(pv, 81, af)
        for j in range(_K):
            gtab[j, pl.ds(g * _L, _L)] = _tree(
                [pv[i * _K + j] * af[i] for i in range(_K)])
        return carry

    lax.fori_loop(0, _APW // _L, build_a, 0)

    def arow(al, carry):
        plo, zvec = carry
        grp = (al // _L) * _L
        off = jnp.full((_L,), al - grp, _i32)
        ea_b = _gtake(eat[pl.ds(grp, _L)], off)
        gb = [_gtake(gtab[j, pl.ds(grp, _L)], off) for j in range(_K)]
        base = _tree([ea_b, _wmask(0) * gb[7], _wmask(1) * gb[8]])
        eac = [_tree([base]
                     + [gb[4 + t] for t in range(3) if (u >> t) & 1])
               for u in range(_NVW)]
        rs0 = zero
        rs1 = zero
        for u in range(_NVW):
            t = _tree([ebv[pl.ds(u * _L, _L)] + eac[u]]
                      + [_lmask(j) * gb[j] for j in range(4)])
            w = jnp.exp(t)
            cacc[pl.ds(u * _L, _L)] = cacc[pl.ds(u * _L, _L)] + w
            if u % 2 == 0:
                rs0 = rs0 + w
            else:
                rs1 = rs1 + w
        rsb = _bsum(rs0 + rs1, lanes)
        zvec = zvec + rsb
        a_full = jnp.full((_L,), a_base + al, _i32)
        a_bits = jnp.where(((a_full >> lanes) & 1) == 1, 1.0, 0.0).astype(_f32)
        return plo + rsb * a_bits, zvec

    plo, zvec = lax.fori_loop(0, _APW, arow, (zero, zero))

    cs = [cacc[pl.ds(u * _L, _L)] for u in range(_NVW)]
    call = _tree(cs)
    phi = zero
    for j in range(4):
        pj = _bsum(_lmask(j) * call, lanes)
        phi = phi + pj * jnp.where(lanes == j, 1.0, 0.0).astype(_f32)
    for t in range(3):
        sub = [cs[u] for u in range(_NVW) if (u >> t) & 1]
        pj = _bsum(_tree(sub), lanes)
        phi = phi + pj * jnp.where(lanes == 4 + t, 1.0, 0.0).astype(_f32)
    for t in range(2):
        pj = _bsum(_wmask(t) * call, lanes)
        phi = phi + pj * jnp.where(lanes == 7 + t, 1.0, 0.0).astype(_f32)

    obuf[pl.ds(0, _L)] = plo
    obuf[pl.ds(_L, _L)] = phi
    obuf[pl.ds(2 * _L, _L)] = zvec
    pltpu.sync_copy(obuf, out_hbm.at[wid])


_mesh = plsc.VectorSubcoreMesh(core_axis_name="c", subcore_axis_name="s",
                               num_cores=_NC, num_subcores=_NS)

_sc_call = functools.partial(
    pl.kernel,
    out_type=jax.ShapeDtypeStruct((_NW, 3 * _L), _f32),
    mesh=_mesh,
    scratch_types=[
        pltpu.VMEM((_PR, _L), _f32),
        pltpu.VMEM((_BPW,), _f32),
        pltpu.VMEM((_BPW,), _f32),
        pltpu.VMEM((_APW,), _f32),
        pltpu.VMEM((_K, _APW), _f32),
        pltpu.VMEM((3 * _L,), _f32),
    ],
)(_sc_body)


def kernel(matrix, beta):
    m = beta * matrix.astype(_f32)
    flat = jnp.concatenate([
        (2.0 * m[:_K, _K:]).reshape(_K * _K),
        m[:_K, :_K].reshape(_K * _K),
        m[_K:, _K:].reshape(_K * _K),
        jnp.zeros((_PR - 3 * _K * _K,), _f32),
    ])
    pb = jnp.broadcast_to(flat[:, None], (_PR, _L))
    out = _sc_call(pb)
    plo = jnp.sum(out[:, :_L], axis=0)
    phi = jnp.sum(out[:, _L:2 * _L], axis=0)
    z = jnp.sum(out[:, 2 * _L])
    prob = jnp.concatenate([plo[:_K], phi[:_K]]) / z
    return prob[None, :_V], prob[None, _V:_N]


# speedup vs baseline: 1.2076x; 1.0771x over previous
import functools

import jax
import jax.numpy as jnp
from jax import lax
from jax.experimental import pallas as pl
from jax.experimental.pallas import tpu as pltpu
from jax.experimental.pallas import tpu_sc as plsc


_K = 9
_S = 1 << _K
_V = 10
_N = 18
_NC = 2
_NS = 16
_NW = _NC * _NS
_L = 16
_AG = 8
_BG = 4
_APW = _S // _AG
_BPW = _S // _BG
_NVW = _BPW // _L
_PF = 336

_f32 = jnp.float32
_i32 = jnp.int32


def _tree(terms):
    terms = list(terms)
    while len(terms) > 1:
        nxt = [terms[i] + terms[i + 1] for i in range(0, len(terms) - 1, 2)]
        if len(terms) % 2:
            nxt.append(terms[-1])
        terms = nxt
    return terms[0]


def _gtake(x, idx):
    return x.at[idx].get(mode="promise_in_bounds")


def _bsum(x, lanes):
    for st in (1, 2, 4, 8):
        x = x + _gtake(x, lanes ^ st)
    return x


def _bits(vec):
    return [jnp.where(((vec >> j) & 1) == 1, 1.0, 0.0).astype(_f32)
            for j in range(_K)]


def _quadform(row, bf):
    return _tree([bf[i] * _tree([row(i, j) * bf[j] for j in range(_K)])
                  for i in range(_K)])


def _sc_body(params_hbm, out_hbm, pvf, ebv, cacc, eat, gtab, obuf):
    c = lax.axis_index("c")
    s = lax.axis_index("s")
    wid = s * _NC + c
    ag = wid // _BG
    bg = wid - ag * _BG
    a_base = ag * _APW
    b_base = bg * _BPW

    pltpu.sync_copy(params_hbm, pvf)

    lanes = lax.iota(_i32, _L)
    zero = jnp.zeros((_L,), _f32)

    def _ment(r, cc):
        k = _N * r + cc
        base = (k // _L) * _L
        return _gtake(pvf[pl.ds(base, _L)], jnp.full((_L,), k - base, _i32))

    def _lmask(j):
        return jnp.where(((lanes >> j) & 1) == 1, 1.0, 0.0).astype(_f32)

    def _wmask(t):
        return jnp.where(((jnp.full((_L,), bg, _i32) >> t) & 1) == 1,
                         1.0, 0.0).astype(_f32)

    def build_v(v, carry):
        bvec = b_base + v * _L + lanes
        bf = _bits(bvec)
        ebv[pl.ds(v * _L, _L)] = _quadform(
            lambda i, j: _ment(_K + i, _K + j), bf)
        cacc[pl.ds(v * _L, _L)] = zero
        return carry

    lax.fori_loop(0, _NVW, build_v, 0)

    def build_a(g, carry):
        avec = a_base + g * _L + lanes
        af = _bits(avec)
        eat[pl.ds(g * _L, _L)] = _quadform(lambda i, j: _ment(i, j), af)
        for j in range(_K):
            gtab[j, pl.ds(g * _L, _L)] = 2.0 * _tree(
                [_ment(i, _K + j) * af[i] for i in range(_K)])
        return carry

    lax.fori_loop(0, _APW // _L, build_a, 0)

    def arow(al, carry):
        plo, zvec = carry
        grp = (al // _L) * _L
        off = jnp.full((_L,), al - grp, _i32)
        ea_b = _gtake(eat[pl.ds(grp, _L)], off)
        gb = [_gtake(gtab[j, pl.ds(grp, _L)], off) for j in range(_K)]
        base = _tree([ea_b, _wmask(0) * gb[7], _wmask(1) * gb[8]])
        eac = [_tree([base]
                     + [gb[4 + t] for t in range(3) if (u >> t) & 1])
               for u in range(_NVW)]
        rs0 = zero
        rs1 = zero
        for u in range(_NVW):
            t = _tree([ebv[pl.ds(u * _L, _L)] + eac[u]]
                      + [_lmask(j) * gb[j] for j in range(4)])
            w = jnp.exp(t)
            cacc[pl.ds(u * _L, _L)] = cacc[pl.ds(u * _L, _L)] + w
            if u % 2 == 0:
                rs0 = rs0 + w
            else:
                rs1 = rs1 + w
        rsb = _bsum(rs0 + rs1, lanes)
        zvec = zvec + rsb
        a_full = jnp.full((_L,), a_base + al, _i32)
        a_bits = jnp.where(((a_full >> lanes) & 1) == 1, 1.0, 0.0).astype(_f32)
        return plo + rsb * a_bits, zvec

    plo, zvec = lax.fori_loop(0, _APW, arow, (zero, zero))

    cs = [cacc[pl.ds(u * _L, _L)] for u in range(_NVW)]
    call = _tree(cs)
    phi = zero
    for j in range(4):
        pj = _bsum(_lmask(j) * call, lanes)
        phi = phi + pj * jnp.where(lanes == j, 1.0, 0.0).astype(_f32)
    for t in range(3):
        sub = [cs[u] for u in range(_NVW) if (u >> t) & 1]
        pj = _bsum(_tree(sub), lanes)
        phi = phi + pj * jnp.where(lanes == 4 + t, 1.0, 0.0).astype(_f32)
    for t in range(2):
        pj = _bsum(_wmask(t) * call, lanes)
        phi = phi + pj * jnp.where(lanes == 7 + t, 1.0, 0.0).astype(_f32)

    obuf[pl.ds(0, _L)] = plo
    obuf[pl.ds(_L, _L)] = phi
    obuf[pl.ds(2 * _L, _L)] = zvec
    pltpu.sync_copy(obuf, out_hbm.at[wid])


_mesh = plsc.VectorSubcoreMesh(core_axis_name="c", subcore_axis_name="s",
                               num_cores=_NC, num_subcores=_NS)

_sc_call = functools.partial(
    pl.kernel,
    out_type=jax.ShapeDtypeStruct((_NW, 3 * _L), _f32),
    mesh=_mesh,
    scratch_types=[
        pltpu.VMEM((_PF,), _f32),
        pltpu.VMEM((_BPW,), _f32),
        pltpu.VMEM((_BPW,), _f32),
        pltpu.VMEM((_APW,), _f32),
        pltpu.VMEM((_K, _APW), _f32),
        pltpu.VMEM((3 * _L,), _f32),
    ],
)(_sc_body)


def kernel(matrix, beta):
    flat = jnp.pad((beta * matrix.astype(_f32)).reshape(_N * _N),
                   (0, _PF - _N * _N))
    out = _sc_call(flat)
    plo = jnp.sum(out[:, :_L], axis=0)
    phi = jnp.sum(out[:, _L:2 * _L], axis=0)
    z = jnp.sum(out[:, 2 * _L])
    prob = jnp.concatenate([plo[:_K], phi[:_K]]) / z
    return prob[None, :_V], prob[None, _V:_N]


# speedup vs baseline: 1.2262x; 1.0154x over previous
import functools

import jax
import jax.numpy as jnp
from jax import lax
from jax.experimental import pallas as pl
from jax.experimental.pallas import tpu as pltpu
from jax.experimental.pallas import tpu_sc as plsc


_K = 9
_S = 1 << _K
_V = 10
_N = 18
_NC = 2
_NS = 16
_NW = _NC * _NS
_L = 16
_AG = 8
_BG = 4
_APW = _S // _AG
_BPW = _S // _BG
_NVW = _BPW // _L
_PF = 336

_f32 = jnp.float32
_i32 = jnp.int32


def _tree(terms):
    terms = list(terms)
    while len(terms) > 1:
        nxt = [terms[i] + terms[i + 1] for i in range(0, len(terms) - 1, 2)]
        if len(terms) % 2:
            nxt.append(terms[-1])
        terms = nxt
    return terms[0]


def _gtake(x, idx):
    return x.at[idx].get(mode="promise_in_bounds")


def _bsum(x, lanes):
    for st in (1, 2, 4, 8):
        x = x + _gtake(x, lanes ^ st)
    return x


def _bits(vec):
    return [jnp.where(((vec >> j) & 1) == 1, 1.0, 0.0).astype(_f32)
            for j in range(_K)]


def _quadform(row, bf):
    return _tree([bf[i] * _tree([row(i, j) * bf[j] for j in range(_K)])
                  for i in range(_K)])


def _sc_body(params_hbm, out_hbm, pvf, ebv, cacc, eat, gtab, obuf):
    c = lax.axis_index("c")
    s = lax.axis_index("s")
    wid = s * _NC + c
    ag = wid // _BG
    bg = wid - ag * _BG
    a_base = ag * _APW
    b_base = bg * _BPW

    pltpu.sync_copy(params_hbm, pvf)

    lanes = lax.iota(_i32, _L)
    zero = jnp.zeros((_L,), _f32)

    def _ment(r, cc):
        k = _N * r + cc
        base = (k // _L) * _L
        return _gtake(pvf[pl.ds(base, _L)], jnp.full((_L,), k - base, _i32))

    def _lmask(j):
        return jnp.where(((lanes >> j) & 1) == 1, 1.0, 0.0).astype(_f32)

    def _wmask(t):
        return jnp.where(((jnp.full((_L,), bg, _i32) >> t) & 1) == 1,
                         1.0, 0.0).astype(_f32)

    def build_v(v, carry):
        bvec = b_base + v * _L + lanes
        bf = _bits(bvec)
        ebv[pl.ds(v * _L, _L)] = _quadform(
            lambda i, j: _ment(_K + i, _K + j), bf)
        cacc[pl.ds(v * _L, _L)] = zero
        return carry

    lax.fori_loop(0, _NVW, build_v, 0)

    def build_a(g, carry):
        avec = a_base + g * _L + lanes
        af = _bits(avec)
        eat[pl.ds(g * _L, _L)] = _quadform(lambda i, j: _ment(i, j), af)
        for j in range(_K):
            gtab[j, pl.ds(g * _L, _L)] = 2.0 * _tree(
                [_ment(i, _K + j) * af[i] for i in range(_K)])
        return carry

    lax.fori_loop(0, _APW // _L, build_a, 0)

    def arow(i2, carry):
        plo, zvec = carry
        al0 = i2 * 2
        grp = (al0 // _L) * _L
        off0 = jnp.full((_L,), al0 - grp, _i32)
        off1 = off0 + 1
        eat_s = eat[pl.ds(grp, _L)]
        ea0 = _gtake(eat_s, off0)
        ea1 = _gtake(eat_s, off1)
        gsl = [gtab[j, pl.ds(grp, _L)] for j in range(_K)]
        gb0 = [_gtake(g, off0) for g in gsl]
        gb1 = [_gtake(g, off1) for g in gsl]
        wm0, wm1 = _wmask(0), _wmask(1)
        base0 = _tree([ea0, wm0 * gb0[7], wm1 * gb0[8]])
        base1 = _tree([ea1, wm0 * gb1[7], wm1 * gb1[8]])
        eac0 = [_tree([base0] + [gb0[4 + t] for t in range(3) if (u >> t) & 1])
                for u in range(_NVW)]
        eac1 = [_tree([base1] + [gb1[4 + t] for t in range(3) if (u >> t) & 1])
                for u in range(_NVW)]
        lm = [_lmask(j) for j in range(4)]
        rsa = zero
        rsb_ = zero
        for u in range(_NVW):
            eb_u = ebv[pl.ds(u * _L, _L)]
            t0 = _tree([eb_u + eac0[u]] + [lm[j] * gb0[j] for j in range(4)])
            t1 = _tree([eb_u + eac1[u]] + [lm[j] * gb1[j] for j in range(4)])
            w0 = jnp.exp(t0)
            w1 = jnp.exp(t1)
            cacc[pl.ds(u * _L, _L)] = cacc[pl.ds(u * _L, _L)] + (w0 + w1)
            rsa = rsa + w0
            rsb_ = rsb_ + w1
        rb0 = _bsum(rsa, lanes)
        rb1 = _bsum(rsb_, lanes)
        zvec = zvec + rb0 + rb1
        af0 = jnp.full((_L,), a_base + al0, _i32)
        ab0 = jnp.where(((af0 >> lanes) & 1) == 1, 1.0, 0.0).astype(_f32)
        ab1 = jnp.where((((af0 + 1) >> lanes) & 1) == 1, 1.0, 0.0).astype(_f32)
        return plo + rb0 * ab0 + rb1 * ab1, zvec

    plo, zvec = lax.fori_loop(0, _APW // 2, arow, (zero, zero))

    cs = [cacc[pl.ds(u * _L, _L)] for u in range(_NVW)]
    call = _tree(cs)
    phi = zero
    for j in range(4):
        pj = _bsum(_lmask(j) * call, lanes)
        phi = phi + pj * jnp.where(lanes == j, 1.0, 0.0).astype(_f32)
    for t in range(3):
        sub = [cs[u] for u in range(_NVW) if (u >> t) & 1]
        pj = _bsum(_tree(sub), lanes)
        phi = phi + pj * jnp.where(lanes == 4 + t, 1.0, 0.0).astype(_f32)
    for t in range(2):
        pj = _bsum(_wmask(t) * call, lanes)
        phi = phi + pj * jnp.where(lanes == 7 + t, 1.0, 0.0).astype(_f32)

    obuf[pl.ds(0, _L)] = plo
    obuf[pl.ds(_L, _L)] = phi
    obuf[pl.ds(2 * _L, _L)] = zvec
    pltpu.sync_copy(obuf, out_hbm.at[wid])


_mesh = plsc.VectorSubcoreMesh(core_axis_name="c", subcore_axis_name="s",
                               num_cores=_NC, num_subcores=_NS)

_sc_call = functools.partial(
    pl.kernel,
    out_type=jax.ShapeDtypeStruct((_NW, 3 * _L), _f32),
    mesh=_mesh,
    scratch_types=[
        pltpu.VMEM((_PF,), _f32),
        pltpu.VMEM((_BPW,), _f32),
        pltpu.VMEM((_BPW,), _f32),
        pltpu.VMEM((_APW,), _f32),
        pltpu.VMEM((_K, _APW), _f32),
        pltpu.VMEM((3 * _L,), _f32),
    ],
)(_sc_body)


def kernel(matrix, beta):
    flat = jnp.pad((beta * matrix.astype(_f32)).reshape(_N * _N),
                   (0, _PF - _N * _N))
    out = _sc_call(flat)
    plo = jnp.sum(out[:, :_L], axis=0)
    phi = jnp.sum(out[:, _L:2 * _L], axis=0)
    z = jnp.sum(out[:, 2 * _L])
    prob = jnp.concatenate([plo[:_K], phi[:_K]]) / z
    return prob[None, :_V], prob[None, _V:_N]
